# Initial kernel scaffold; baseline (speedup 1.0000x reference)
import jax
import jax.numpy as jnp
from jax.experimental import pallas as pl


def kernel(x, edge_index, W1, b1, W2, b2):
    raise NotImplementedError("write your pallas kernel here")


# speedup vs baseline: 4.3336x; 4.3336x over previous
import functools
import jax
import jax.numpy as jnp
from jax import lax
from jax.experimental import pallas as pl
from jax.experimental.pallas import tpu as pltpu
from jax.experimental.pallas import tpu_sc as plsc

NN = 10000
NP = 10240
D = 128
KHOPS = 3
NC = 2
NS = 16
RPT = NP // NS
EP = 327680
CHUNK = 1024
NCHUNKS = EP // CHUNK
CPT = NCHUNKS // (NC * NS)

_mesh = plsc.VectorSubcoreMesh(core_axis_name="c", subcore_axis_name="s",
                               num_cores=NC, num_subcores=NS)


@functools.partial(
    pl.kernel,
    out_type=jax.ShapeDtypeStruct((NC, NP), jnp.float32),
    mesh=_mesh,
    scratch_types=[
        pltpu.VMEM((8, 128), jnp.int32),
        pltpu.VMEM((128,), jnp.float32),
        pltpu.VMEM_SHARED((NP,), jnp.float32),
        pltpu.SemaphoreType.DMA,
    ],
)
def _deg_kernel(col2d, zeros1, deg_out, cidx, ones_v, acc, sem):
    del sem
    cid = lax.axis_index("c")
    sid = lax.axis_index("s")
    for j in range(8):
        ones_v[pl.ds(j * 16, 16)] = jnp.ones((16,), jnp.float32)
    pltpu.sync_copy(zeros1.at[pl.ds(sid * RPT, RPT)],
                    acc.at[pl.ds(sid * RPT, RPT)])
    plsc.subcore_barrier()
    wid = cid * NS + sid

    def body(ch, carry):
        g = wid * CPT + ch
        pltpu.sync_copy(col2d.at[pl.ds(g * 8, 8)], cidx)
        for j in range(8):
            pltpu.sync_copy(ones_v, acc.at[cidx.at[j]], add=True)
        return carry

    lax.fori_loop(0, CPT, body, 0)
    plsc.subcore_barrier()
    pltpu.sync_copy(acc.at[pl.ds(sid * RPT, RPT)],
                    deg_out.at[cid, pl.ds(sid * RPT, RPT)])


@functools.partial(
    pl.kernel,
    out_type=jax.ShapeDtypeStruct((NC, NP, D), jnp.float32),
    mesh=_mesh,
    scratch_types=[
        pltpu.VMEM((8, 128), jnp.int32),
        pltpu.VMEM((8, 128), jnp.int32),
        pltpu.VMEM((128, D), jnp.float32),
        pltpu.VMEM_SHARED((NP, D), jnp.float32),
        pltpu.SemaphoreType.DMA,
    ],
)
def _prop_kernel(u, row2d, col2d, zeros2, s_out, ridx, cidx, gbuf, acc, sem):
    cid = lax.axis_index("c")
    sid = lax.axis_index("s")
    pltpu.sync_copy(zeros2.at[pl.ds(sid * RPT, RPT)],
                    acc.at[pl.ds(sid * RPT, RPT)])
    plsc.subcore_barrier()
    wid = cid * NS + sid

    def body(ch, carry):
        g = wid * CPT + ch
        pltpu.sync_copy(row2d.at[pl.ds(g * 8, 8)], ridx)
        pltpu.sync_copy(col2d.at[pl.ds(g * 8, 8)], cidx)
        for j in range(8):
            pltpu.async_copy(u.at[ridx.at[j]], gbuf, sem).wait()
            pltpu.sync_copy(gbuf, acc.at[cidx.at[j]], add=True)
        return carry

    lax.fori_loop(0, CPT, body, 0)
    plsc.subcore_barrier()
    pltpu.sync_copy(acc.at[pl.ds(sid * RPT, RPT)],
                    s_out.at[cid, pl.ds(sid * RPT, RPT)])


_HI = lax.Precision.HIGHEST


def _prep_body(dega, degb, x, dis, dis2, u):
    d = dega[...] + degb[...]
    r = jnp.where(d > 0, lax.rsqrt(jnp.maximum(d, 1e-12)), 0.0)
    dis[...] = r
    dis2[...] = r * r
    u[...] = r * x[...]


def _prep(deg2, x):
    dega = deg2[0].reshape(NP, 1)
    degb = deg2[1].reshape(NP, 1)
    grid = (NP // 128,)
    return pl.pallas_call(
        _prep_body,
        grid=grid,
        in_specs=[
            pl.BlockSpec((128, 1), lambda i: (i, 0)),
            pl.BlockSpec((128, 1), lambda i: (i, 0)),
            pl.BlockSpec((128, D), lambda i: (i, 0)),
        ],
        out_specs=[
            pl.BlockSpec((128, 1), lambda i: (i, 0)),
            pl.BlockSpec((128, 1), lambda i: (i, 0)),
            pl.BlockSpec((128, D), lambda i: (i, 0)),
        ],
        out_shape=[
            jax.ShapeDtypeStruct((NP, 1), jnp.float32),
            jax.ShapeDtypeStruct((NP, 1), jnp.float32),
            jax.ShapeDtypeStruct((NP, D), jnp.float32),
        ],
    )(dega, degb, x)


def _uscale_body(s, dis2, u):
    sb = s[...]
    u[...] = dis2[...] * (sb[0] + sb[1])


def _uscale(s, dis2):
    grid = (NP // 128,)
    return pl.pallas_call(
        _uscale_body,
        grid=grid,
        in_specs=[
            pl.BlockSpec((NC, 128, D), lambda i: (0, i, 0)),
            pl.BlockSpec((128, 1), lambda i: (i, 0)),
        ],
        out_specs=pl.BlockSpec((128, D), lambda i: (i, 0)),
        out_shape=jax.ShapeDtypeStruct((NP, D), jnp.float32),
    )(s, dis2)


def _matmul_body(relu_u, x, s1, s2, s3, dis, w, b, out, u=None):
    dv = dis[...]
    acc = jnp.dot(x[...], w[0], precision=_HI)
    for k, s in enumerate((s1, s2, s3)):
        sb = s[...]
        h = dv * (sb[0] + sb[1])
        acc = acc + jnp.dot(h, w[k + 1], precision=_HI)
    acc = acc + b[...]
    if relu_u:
        acc = jnp.maximum(acc, 0.0)
        u[...] = dv * acc
    out[...] = acc


def _matmul(x, s1, s2, s3, dis, w, b, relu_u):
    grid = (NP // 128,)
    sspec = pl.BlockSpec((NC, 128, D), lambda i: (0, i, 0))
    out_shape = [jax.ShapeDtypeStruct((NP, D), jnp.float32)]
    out_specs = [pl.BlockSpec((128, D), lambda i: (i, 0))]
    if relu_u:
        out_shape.append(jax.ShapeDtypeStruct((NP, D), jnp.float32))
        out_specs.append(pl.BlockSpec((128, D), lambda i: (i, 0)))
    return pl.pallas_call(
        functools.partial(_matmul_body, relu_u),
        grid=grid,
        in_specs=[
            pl.BlockSpec((128, D), lambda i: (i, 0)),
            sspec, sspec, sspec,
            pl.BlockSpec((128, 1), lambda i: (i, 0)),
            pl.BlockSpec((KHOPS + 1, D, D), lambda i: (0, 0, 0)),
            pl.BlockSpec((1, D), lambda i: (0, 0)),
        ],
        out_specs=out_specs,
        out_shape=out_shape,
    )(x, s1, s2, s3, dis, w, b)


def kernel(x, edge_index, W1, b1, W2, b2):
    x = jnp.pad(x, ((0, NP - NN), (0, 0)))
    npad = EP - edge_index.shape[1]
    pad = jnp.full((2, npad), NP - 1, dtype=jnp.int32)
    ei = jnp.concatenate([edge_index.astype(jnp.int32), pad], axis=1)
    row2d = ei[0].reshape(EP // 128, 128)
    col2d = ei[1].reshape(EP // 128, 128)
    zeros1 = jnp.zeros((NP,), jnp.float32)
    zeros2 = jnp.zeros((NP, D), jnp.float32)
    b1r = b1.reshape(1, D)
    b2r = b2.reshape(1, D)

    deg2 = _deg_kernel(col2d, zeros1)
    dis, dis2, u = _prep(deg2, x)

    def layer(xin, u, w, br, relu_u):
        ss = []
        for k in range(KHOPS):
            s = _prop_kernel(u, row2d, col2d, zeros2)
            ss.append(s)
            if k + 1 < KHOPS:
                u = _uscale(s, dis2)
        return _matmul(xin, ss[0], ss[1], ss[2], dis, w, br, relu_u)

    x2, u2 = layer(x, u, W1, b1r, True)
    (out,) = layer(x2, u2, W2, b2r, False)
    return out[:NN].reshape(-1)
